# baseline (speedup 1.0000x reference)
import functools
import math

import jax
import jax.numpy as jnp
from jax import lax
from jax.experimental import pallas as pl
from jax.experimental.pallas import tpu as pltpu
from jax.experimental.pallas import tpu_sc as plsc

N = 5000
NPAD = 5120
KOSC = 28
KP = 32
BM = 512
THRESH = 0.3
EPS = 1e-6
TWO_PI = 2.0 * math.pi

NW = 32
NSETS = 2
COLS_W = NPAD // NW
LSTRIDE = COLS_W + 1
NBEST = NSETS * 16 * LSTRIDE
NBEST_PAD = (NBEST + 63) // 64 * 64
ROWS_W = NPAD // NW
NCHUNK = NPAD // 16
BIG = 1 << 30


BMP = 1024


def _phase_body(dt_ref, dt1_ref, w1_ref, b1_ref, w2_ref, b2_ref, fr_ref, ph_ref):
    is_t = pl.program_id(0) < NPAD // BMP
    d = jnp.where(is_t, dt_ref[...], dt1_ref[...])
    hT = jax.nn.relu(
        jax.lax.dot_general(w1_ref[...], d, (((0,), (1,)), ((), ())))
        + b1_ref[...])
    phase = (jax.lax.dot_general(w2_ref[...], hT, (((0,), (0,)), ((), ())))
             + b2_ref[...]) % TWO_PI
    fr = fr_ref[...]
    evolved = phase
    for _ in range(5):
        evolved = (evolved + TWO_PI * fr * 0.01) % TWO_PI
    ph_ref[...] = jnp.where(is_t, evolved, phase)


_NBLK_T = NPAD // BMP

_phase_call = pl.pallas_call(
    _phase_body,
    grid=(2 * NPAD // BMP,),
    in_specs=[
        pl.BlockSpec((BMP, 4), lambda i: (jnp.minimum(i, _NBLK_T - 1), 0)),
        pl.BlockSpec((BMP, 4),
                     lambda i: (jnp.clip(i - _NBLK_T, 0, _NBLK_T - 1), 0)),
        pl.BlockSpec((4, 64), lambda i: (0, 0)),
        pl.BlockSpec((64, 1), lambda i: (0, 0)),
        pl.BlockSpec((64, KP), lambda i: (0, 0)),
        pl.BlockSpec((KP, 1), lambda i: (0, 0)),
        pl.BlockSpec((KP, 1), lambda i: (0, 0)),
    ],
    out_specs=pl.BlockSpec((KP, BMP), lambda i: (0, i)),
    out_shape=jax.ShapeDtypeStruct((KP, 2 * NPAD), jnp.float32),
)


def _sim_body(ac_ref, as_ref, bc_ref, bs_ref, sim_ref, mx_ref, mi_ref):
    dn = (((0,), (0,)), ((), ()))
    s = (jax.lax.dot_general(ac_ref[...], bc_ref[...], dn)
         + jax.lax.dot_general(as_ref[...], bs_ref[...], dn))
    sim_ref[...] = s
    col = jax.lax.broadcasted_iota(jnp.int32, (BM, NPAD), 1)
    sm = jnp.where(col < N, s, -jnp.inf)
    mx = jnp.max(sm, axis=1)
    mi = jnp.min(jnp.where(sm == mx[:, None], col, 2 ** 30), axis=1)
    mx_ref[...] = mx
    mi_ref[...] = mi


_sim_call = pl.pallas_call(
    _sim_body,
    grid=(NPAD // BM,),
    in_specs=[
        pl.BlockSpec((KP, BM), lambda i: (0, i)),
        pl.BlockSpec((KP, BM), lambda i: (0, i)),
        pl.BlockSpec((KP, NPAD), lambda i: (0, 1)),
        pl.BlockSpec((KP, NPAD), lambda i: (0, 1)),
    ],
    out_specs=[
        pl.BlockSpec((BM, NPAD), lambda i: (i, 0)),
        pl.BlockSpec((BM,), lambda i: (i,)),
        pl.BlockSpec((BM,), lambda i: (i,)),
    ],
    out_shape=[
        jax.ShapeDtypeStruct((N, N), jnp.float32),
        jax.ShapeDtypeStruct((NPAD,), jnp.float32),
        jax.ShapeDtypeStruct((NPAD,), jnp.int32),
    ],
)


_SC_MESH = plsc.VectorSubcoreMesh(core_axis_name="c", subcore_axis_name="s")


def _match_a_body(sims_hbm, idxs_hbm, colwin_hbm, sims_v, idxs_v, bestk_v,
                  besti_v, colwin_v):
    w = lax.axis_index("s") * 2 + lax.axis_index("c")
    lo = w * COLS_W
    pltpu.sync_copy(sims_hbm, sims_v)
    pltpu.sync_copy(idxs_hbm, idxs_v)

    zero16 = jnp.zeros((16,), jnp.float32)
    big16 = jnp.full((16,), BIG, jnp.int32)

    def init(c, _):
        for u in range(4):
            bestk_v[pl.ds(c * 64 + u * 16, 16)] = zero16
            besti_v[pl.ds(c * 64 + u * 16, 16)] = big16
        return 0

    lax.fori_loop(0, NBEST_PAD // 64, init, 0)

    iota16 = lax.iota(jnp.int32, 16)
    lane_base = iota16 * LSTRIDE

    def pass_a(u, _):
        for s in range(NSETS):
            t = u * NSETS + s
            sv = sims_v[pl.ds(t * 16, 16)]
            jv = idxs_v[pl.ds(t * 16, 16)]
            key = jnp.where(sv > THRESH, sv, 0.0)
            jl = jnp.clip(jv - lo, 0, COLS_W - 1)
            owned = (jv >= lo) & (jv < lo + COLS_W) & (key > 0.0)
            rowid = t * 16 + iota16
            flat = s * (16 * LSTRIDE) + lane_base + jl
            cur_k = plsc.load_gather(bestk_v, [flat])
            cur_i = plsc.load_gather(besti_v, [flat])
            winm = owned & (key > cur_k)
            plsc.store_scatter(bestk_v, [flat], jnp.where(winm, key, cur_k))
            plsc.store_scatter(besti_v, [flat], jnp.where(winm, rowid, cur_i))
        return 0

    lax.fori_loop(0, NCHUNK // NSETS, pass_a, 0)

    def merge(c, _):
        acc_k = bestk_v[pl.ds(c * 16, 16)]
        acc_i = besti_v[pl.ds(c * 16, 16)]
        for sl in range(1, NSETS * 16):
            k = bestk_v[pl.ds(sl * LSTRIDE + c * 16, 16)]
            i = besti_v[pl.ds(sl * LSTRIDE + c * 16, 16)]
            better = (k > acc_k) | ((k == acc_k) & (i < acc_i))
            acc_k = jnp.where(better, k, acc_k)
            acc_i = jnp.where(better, i, acc_i)
        colwin_v[pl.ds(c * 16, 16)] = acc_i
        return 0

    lax.fori_loop(0, COLS_W // 16, merge, 0)
    pltpu.sync_copy(colwin_v, colwin_hbm.at[pl.ds(lo, COLS_W)])


_match_a = functools.partial(
    pl.kernel,
    out_type=jax.ShapeDtypeStruct((NPAD,), jnp.int32),
    mesh=_SC_MESH,
    compiler_params=pltpu.CompilerParams(needs_layout_passes=False),
    scratch_types=[
        pltpu.VMEM((NPAD,), jnp.float32),
        pltpu.VMEM((NPAD,), jnp.int32),
        pltpu.VMEM((NBEST_PAD,), jnp.float32),
        pltpu.VMEM((NBEST_PAD,), jnp.int32),
        pltpu.VMEM((COLS_W,), jnp.int32),
    ],
)(_match_a_body)


def _match_b_body(sims_hbm, idxs_hbm, colwin_hbm, out_hbm, sims_v, idxs_v,
                  colwin_v, match_v):
    w = lax.axis_index("s") * 2 + lax.axis_index("c")
    rlo = w * ROWS_W
    pltpu.sync_copy(colwin_hbm, colwin_v)
    pltpu.sync_copy(sims_hbm.at[pl.ds(rlo, ROWS_W)], sims_v)
    pltpu.sync_copy(idxs_hbm.at[pl.ds(rlo, ROWS_W)], idxs_v)

    iota16 = lax.iota(jnp.int32, 16)
    neg16 = jnp.full((16,), -1, jnp.int32)

    def rowph(t, _):
        sv = sims_v[pl.ds(t * 16, 16)]
        jv = idxs_v[pl.ds(t * 16, 16)]
        rowid = rlo + t * 16 + iota16
        winner = plsc.load_gather(colwin_v, [jv])
        ok = (sv > THRESH) & (winner == rowid)
        match_v[pl.ds(t * 16, 16)] = jnp.where(ok, jv, neg16)
        return 0

    lax.fori_loop(0, ROWS_W // 16, rowph, 0)
    pltpu.sync_copy(match_v, out_hbm.at[pl.ds(rlo, ROWS_W)])


_match_b = functools.partial(
    pl.kernel,
    out_type=jax.ShapeDtypeStruct((NPAD,), jnp.int32),
    mesh=_SC_MESH,
    compiler_params=pltpu.CompilerParams(needs_layout_passes=False),
    scratch_types=[
        pltpu.VMEM((ROWS_W,), jnp.float32),
        pltpu.VMEM((ROWS_W,), jnp.int32),
        pltpu.VMEM((NPAD,), jnp.int32),
        pltpu.VMEM((ROWS_W,), jnp.int32),
    ],
)(_match_b_body)


def kernel(detections_t, detections_t1, Wp1, bp1, Wp2, bp2, Wa1, ba1, Wa2, ba2):
    del Wa1, ba1, Wa2, ba2

    b1 = bp1[:, None]
    w2 = jnp.pad(Wp2, ((0, 0), (0, KP - KOSC)))
    b2 = jnp.pad(bp2, (0, KP - KOSC))[:, None]
    freqs = jnp.concatenate([
        jnp.full((4,), 2.0, jnp.float32),
        jnp.full((8,), 6.0, jnp.float32),
        jnp.full((16,), 40.0, jnp.float32),
    ])
    fr = jnp.pad(freqs, (0, KP - KOSC))[:, None]

    ph = _phase_call(detections_t, detections_t1, Wp1, b1, w2, b2, fr)

    osc_ok = jax.lax.broadcasted_iota(jnp.int32, (KP, 1), 0) < KOSC
    col = jax.lax.broadcasted_iota(jnp.int32, (1, 2 * NPAD), 1)
    row_ok = (col < N) | ((col >= NPAD) & (col < NPAD + N))
    mask = osc_ok & row_ok
    re = jnp.where(mask, jnp.cos(ph), 0.0)
    im = jnp.where(mask, jnp.sin(ph), 0.0)
    nrm = jnp.sqrt(jnp.sum(re * re + im * im, axis=0, keepdims=True)) + EPS
    ra = re / nrm
    ia = im / nrm

    sim, mx, mi = _sim_call(ra, ia, ra, ia)

    colwin = _match_a(mx, mi)
    matches_p = _match_b(mx, mi, colwin)
    return matches_p[:N], sim


# speedup vs baseline: 1.0098x; 1.0098x over previous
import functools
import math

import jax
import jax.numpy as jnp
from jax import lax
from jax.experimental import pallas as pl
from jax.experimental.pallas import tpu as pltpu
from jax.experimental.pallas import tpu_sc as plsc

N = 5000
NPAD = 5120
KOSC = 28
KP = 32
BM = 1024
THRESH = 0.3
EPS = 1e-6
TWO_PI = 2.0 * math.pi

NW = 32
NSETS = 2
COLS_W = NPAD // NW
LSTRIDE = COLS_W + 1
NBEST = NSETS * 16 * LSTRIDE
NBEST_PAD = (NBEST + 63) // 64 * 64
ROWS_W = NPAD // NW
NCHUNK = NPAD // 16
BIG = 1 << 30


BMP = 2560


def _phase_body(dt_ref, dt1_ref, w1_ref, b1_ref, w2_ref, b2_ref, fr_ref, ph_ref):
    is_t = pl.program_id(0) < NPAD // BMP
    d = jnp.where(is_t, dt_ref[...], dt1_ref[...])
    hT = jax.nn.relu(
        jax.lax.dot_general(w1_ref[...], d, (((0,), (1,)), ((), ())))
        + b1_ref[...])
    phase = (jax.lax.dot_general(w2_ref[...], hT, (((0,), (0,)), ((), ())))
             + b2_ref[...]) % TWO_PI
    fr = fr_ref[...]
    evolved = phase
    for _ in range(5):
        evolved = (evolved + TWO_PI * fr * 0.01) % TWO_PI
    ph_ref[...] = jnp.where(is_t, evolved, phase)


_NBLK_T = NPAD // BMP

_phase_call = pl.pallas_call(
    _phase_body,
    grid=(2 * NPAD // BMP,),
    in_specs=[
        pl.BlockSpec((BMP, 4), lambda i: (jnp.minimum(i, _NBLK_T - 1), 0)),
        pl.BlockSpec((BMP, 4),
                     lambda i: (jnp.clip(i - _NBLK_T, 0, _NBLK_T - 1), 0)),
        pl.BlockSpec((4, 64), lambda i: (0, 0)),
        pl.BlockSpec((64, 1), lambda i: (0, 0)),
        pl.BlockSpec((64, KP), lambda i: (0, 0)),
        pl.BlockSpec((KP, 1), lambda i: (0, 0)),
        pl.BlockSpec((KP, 1), lambda i: (0, 0)),
    ],
    out_specs=pl.BlockSpec((KP, BMP), lambda i: (0, i)),
    out_shape=jax.ShapeDtypeStruct((KP, 2 * NPAD), jnp.float32),
)


def _sim_body(ac_ref, as_ref, bc_ref, bs_ref, sim_ref, mx_ref, mi_ref):
    dn = (((0,), (0,)), ((), ()))
    s = (jax.lax.dot_general(ac_ref[...], bc_ref[...], dn)
         + jax.lax.dot_general(as_ref[...], bs_ref[...], dn))
    sim_ref[...] = s
    col = jax.lax.broadcasted_iota(jnp.int32, (BM, NPAD), 1)
    sm = jnp.where(col < N, s, -jnp.inf)
    mx = jnp.max(sm, axis=1)
    mi = jnp.min(jnp.where(sm == mx[:, None], col, 2 ** 30), axis=1)
    mx_ref[...] = mx
    mi_ref[...] = mi


_sim_call = pl.pallas_call(
    _sim_body,
    grid=(NPAD // BM,),
    in_specs=[
        pl.BlockSpec((KP, BM), lambda i: (0, i)),
        pl.BlockSpec((KP, BM), lambda i: (0, i)),
        pl.BlockSpec((KP, NPAD), lambda i: (0, 1)),
        pl.BlockSpec((KP, NPAD), lambda i: (0, 1)),
    ],
    out_specs=[
        pl.BlockSpec((BM, NPAD), lambda i: (i, 0)),
        pl.BlockSpec((BM,), lambda i: (i,)),
        pl.BlockSpec((BM,), lambda i: (i,)),
    ],
    out_shape=[
        jax.ShapeDtypeStruct((N, N), jnp.float32),
        jax.ShapeDtypeStruct((NPAD,), jnp.float32),
        jax.ShapeDtypeStruct((NPAD,), jnp.int32),
    ],
    compiler_params=pltpu.CompilerParams(vmem_limit_bytes=100 * 1024 * 1024),
)


_SC_MESH = plsc.VectorSubcoreMesh(core_axis_name="c", subcore_axis_name="s")


def _match_a_body(sims_hbm, idxs_hbm, colwin_hbm, sims_v, idxs_v, bestk_v,
                  besti_v, colwin_v):
    w = lax.axis_index("s") * 2 + lax.axis_index("c")
    lo = w * COLS_W
    pltpu.sync_copy(sims_hbm, sims_v)
    pltpu.sync_copy(idxs_hbm, idxs_v)

    zero16 = jnp.zeros((16,), jnp.float32)
    big16 = jnp.full((16,), BIG, jnp.int32)

    def init(c, _):
        for u in range(4):
            bestk_v[pl.ds(c * 64 + u * 16, 16)] = zero16
            besti_v[pl.ds(c * 64 + u * 16, 16)] = big16
        return 0

    lax.fori_loop(0, NBEST_PAD // 64, init, 0)

    iota16 = lax.iota(jnp.int32, 16)
    lane_base = iota16 * LSTRIDE

    def pass_a(u, _):
        for s in range(NSETS):
            t = u * NSETS + s
            sv = sims_v[pl.ds(t * 16, 16)]
            jv = idxs_v[pl.ds(t * 16, 16)]
            key = jnp.where(sv > THRESH, sv, 0.0)
            jl = jnp.clip(jv - lo, 0, COLS_W - 1)
            owned = (jv >= lo) & (jv < lo + COLS_W) & (key > 0.0)
            rowid = t * 16 + iota16
            flat = s * (16 * LSTRIDE) + lane_base + jl
            cur_k = plsc.load_gather(bestk_v, [flat])
            cur_i = plsc.load_gather(besti_v, [flat])
            winm = owned & (key > cur_k)
            plsc.store_scatter(bestk_v, [flat], jnp.where(winm, key, cur_k))
            plsc.store_scatter(besti_v, [flat], jnp.where(winm, rowid, cur_i))
        return 0

    lax.fori_loop(0, NCHUNK // NSETS, pass_a, 0)

    def merge(c, _):
        acc_k = bestk_v[pl.ds(c * 16, 16)]
        acc_i = besti_v[pl.ds(c * 16, 16)]
        for sl in range(1, NSETS * 16):
            k = bestk_v[pl.ds(sl * LSTRIDE + c * 16, 16)]
            i = besti_v[pl.ds(sl * LSTRIDE + c * 16, 16)]
            better = (k > acc_k) | ((k == acc_k) & (i < acc_i))
            acc_k = jnp.where(better, k, acc_k)
            acc_i = jnp.where(better, i, acc_i)
        colwin_v[pl.ds(c * 16, 16)] = acc_i
        return 0

    lax.fori_loop(0, COLS_W // 16, merge, 0)
    pltpu.sync_copy(colwin_v, colwin_hbm.at[pl.ds(lo, COLS_W)])


_match_a = functools.partial(
    pl.kernel,
    out_type=jax.ShapeDtypeStruct((NPAD,), jnp.int32),
    mesh=_SC_MESH,
    compiler_params=pltpu.CompilerParams(needs_layout_passes=False),
    scratch_types=[
        pltpu.VMEM((NPAD,), jnp.float32),
        pltpu.VMEM((NPAD,), jnp.int32),
        pltpu.VMEM((NBEST_PAD,), jnp.float32),
        pltpu.VMEM((NBEST_PAD,), jnp.int32),
        pltpu.VMEM((COLS_W,), jnp.int32),
    ],
)(_match_a_body)


def _match_b_body(sims_hbm, idxs_hbm, colwin_hbm, out_hbm, sims_v, idxs_v,
                  colwin_v, match_v):
    w = lax.axis_index("s") * 2 + lax.axis_index("c")
    rlo = w * ROWS_W
    pltpu.sync_copy(colwin_hbm, colwin_v)
    pltpu.sync_copy(sims_hbm.at[pl.ds(rlo, ROWS_W)], sims_v)
    pltpu.sync_copy(idxs_hbm.at[pl.ds(rlo, ROWS_W)], idxs_v)

    iota16 = lax.iota(jnp.int32, 16)
    neg16 = jnp.full((16,), -1, jnp.int32)

    def rowph(t, _):
        sv = sims_v[pl.ds(t * 16, 16)]
        jv = idxs_v[pl.ds(t * 16, 16)]
        rowid = rlo + t * 16 + iota16
        winner = plsc.load_gather(colwin_v, [jv])
        ok = (sv > THRESH) & (winner == rowid)
        match_v[pl.ds(t * 16, 16)] = jnp.where(ok, jv, neg16)
        return 0

    lax.fori_loop(0, ROWS_W // 16, rowph, 0)
    pltpu.sync_copy(match_v, out_hbm.at[pl.ds(rlo, ROWS_W)])


_match_b = functools.partial(
    pl.kernel,
    out_type=jax.ShapeDtypeStruct((NPAD,), jnp.int32),
    mesh=_SC_MESH,
    compiler_params=pltpu.CompilerParams(needs_layout_passes=False),
    scratch_types=[
        pltpu.VMEM((ROWS_W,), jnp.float32),
        pltpu.VMEM((ROWS_W,), jnp.int32),
        pltpu.VMEM((NPAD,), jnp.int32),
        pltpu.VMEM((ROWS_W,), jnp.int32),
    ],
)(_match_b_body)


def kernel(detections_t, detections_t1, Wp1, bp1, Wp2, bp2, Wa1, ba1, Wa2, ba2):
    del Wa1, ba1, Wa2, ba2

    b1 = bp1[:, None]
    w2 = jnp.pad(Wp2, ((0, 0), (0, KP - KOSC)))
    b2 = jnp.pad(bp2, (0, KP - KOSC))[:, None]
    freqs = jnp.concatenate([
        jnp.full((4,), 2.0, jnp.float32),
        jnp.full((8,), 6.0, jnp.float32),
        jnp.full((16,), 40.0, jnp.float32),
    ])
    fr = jnp.pad(freqs, (0, KP - KOSC))[:, None]

    ph = _phase_call(detections_t, detections_t1, Wp1, b1, w2, b2, fr)

    osc_ok = jax.lax.broadcasted_iota(jnp.int32, (KP, 1), 0) < KOSC
    col = jax.lax.broadcasted_iota(jnp.int32, (1, 2 * NPAD), 1)
    row_ok = (col < N) | ((col >= NPAD) & (col < NPAD + N))
    mask = osc_ok & row_ok
    re = jnp.where(mask, jnp.cos(ph), 0.0)
    im = jnp.where(mask, jnp.sin(ph), 0.0)
    nrm = jnp.sqrt(jnp.sum(re * re + im * im, axis=0, keepdims=True)) + EPS
    ra = re / nrm
    ia = im / nrm

    sim, mx, mi = _sim_call(ra, ia, ra, ia)

    colwin = _match_a(mx, mi)
    matches_p = _match_b(mx, mi, colwin)
    return matches_p[:N], sim
